# Initial kernel scaffold; baseline (speedup 1.0000x reference)
import jax
import jax.numpy as jnp
from jax.experimental import pallas as pl


def kernel(x, edge_index_list, W, b):
    raise NotImplementedError("write your pallas kernel here")


# speedup vs baseline: 6.7550x; 6.7550x over previous
import functools

import jax
import jax.numpy as jnp
from jax import lax
from jax.experimental import pallas as pl
from jax.experimental.pallas import tpu as pltpu
from jax.experimental.pallas import tpu_sc as plsc

N = 10000
NP = 10240
E = 160000
R = 3
D = 256
DH = 128
NC = 2
NS = 16
CH = 128
NCH = 80
G = 8
NG = NCH // G
EPT = CH * NCH
EPAD = NS * EPT
ROWS_PT = NP // NS
BLK = 2048
GRID = NP // BLK

_mesh = plsc.VectorSubcoreMesh(
    core_axis_name="c", subcore_axis_name="s", num_cores=NC, num_subcores=NS)


RPT = NP // NS


def _deg_body(d0, d1, d2, deg_out, hist_sh, idx_v, hist_v, comb_v, out_v):
    c = lax.axis_index("c")
    s = lax.axis_index("s")
    zeros16 = jnp.zeros((16,), jnp.float32)
    ones16 = jnp.ones((16,), jnp.float32)
    zidx16 = jnp.zeros((16,), jnp.int32)

    def do_rel(dref, r):
        pltpu.sync_copy(dref.at[s], idx_v)

        def zero(i, carry):
            hist_v[0, pl.ds(i * 16, 16)] = zeros16
            return carry

        lax.fori_loop(0, NP // 16, zero, 0)

        def acc_chunk(j, carry):
            def acc(k, carry2):
                iv = idx_v[j, pl.ds(k * 16, 16)]
                plsc.addupdate_scatter(hist_v, [zidx16, iv], ones16)
                return carry2

            return lax.fori_loop(0, CH // 16, acc, carry)

        lax.fori_loop(0, NCH, acc_chunk, 0)
        pltpu.sync_copy(hist_v, hist_sh.at[pl.ds(s, 1), :])
        plsc.subcore_barrier()
        pltpu.sync_copy(hist_sh.at[:, pl.ds(s * RPT, RPT)], comb_v)

        def comb(i, carry):
            tot = comb_v[0, pl.ds(i * 16, 16)]
            for t in range(1, NS):
                tot = tot + comb_v[t, pl.ds(i * 16, 16)]
            out_v[0, pl.ds(i * 16, 16)] = tot
            return carry

        lax.fori_loop(0, RPT // 16, comb, 0)
        pltpu.sync_copy(out_v, deg_out.at[pl.ds(r, 1), pl.ds(s * RPT, RPT)])
        plsc.subcore_barrier()

    @pl.when(c == 0)
    def _():
        do_rel(d0, 0)
        do_rel(d1, 1)

    @pl.when(c == 1)
    def _():
        do_rel(d2, 2)


_deg_call = functools.partial(
    pl.kernel,
    out_type=jax.ShapeDtypeStruct((R, NP), jnp.float32),
    mesh=_mesh,
    compiler_params=pltpu.CompilerParams(needs_layout_passes=False),
    scratch_types=[
        pltpu.VMEM_SHARED((NS, NP), jnp.float32),
        pltpu.VMEM((NCH, CH), jnp.int32),
        pltpu.VMEM((1, NP), jnp.float32),
        pltpu.VMEM((NS, RPT), jnp.float32),
        pltpu.VMEM((1, RPT), jnp.float32),
    ],
)(_deg_body)


def _conv_body(h00, h01, h10, h11, h20, h21,
               s0, s1, s2, d0, d1, d2,
               o00, o01, o10, o11, o20, o21,
               acc_sh, src_v, dst_v, buf0, buf1, sem0, sem1):
    c = lax.axis_index("c")
    s = lax.axis_index("s")
    h_refs = ((h00, h01), (h10, h11), (h20, h21))
    o_refs = ((o00, o01), (o10, o11), (o20, o21))
    s_refs = (s0, s1, s2)
    d_refs = (d0, d1, d2)

    def do_half(half):
        for r in range(R):
            h = h_refs[r][half]
            o = o_refs[r][half]
            pltpu.sync_copy(h.at[pl.ds(s * ROWS_PT, ROWS_PT)],
                            acc_sh.at[pl.ds(s * ROWS_PT, ROWS_PT)])
            plsc.subcore_barrier()

            def group(g, carry):
                pltpu.sync_copy(s_refs[r].at[s, pl.ds(g * G, G)], src_v)
                pltpu.sync_copy(d_refs[r].at[s, pl.ds(g * G, G)], dst_v)

                def body(j, carry2):
                    i0 = 2 * j
                    cp0 = pltpu.async_copy(h.at[src_v.at[i0]], buf0, sem0)
                    cp1 = pltpu.async_copy(h.at[src_v.at[i0 + 1]], buf1, sem1)
                    cp0.wait()
                    pltpu.sync_copy(buf0, acc_sh.at[dst_v.at[i0]], add=True)
                    cp1.wait()
                    pltpu.sync_copy(buf1, acc_sh.at[dst_v.at[i0 + 1]],
                                    add=True)
                    return carry2

                return lax.fori_loop(0, G // 2, body, carry)

            lax.fori_loop(0, NG, group, 0)
            plsc.subcore_barrier()
            pltpu.sync_copy(acc_sh.at[pl.ds(s * ROWS_PT, ROWS_PT)],
                            o.at[pl.ds(s * ROWS_PT, ROWS_PT)])
            plsc.subcore_barrier()

    @pl.when(c == 0)
    def _():
        do_half(0)

    @pl.when(c == 1)
    def _():
        do_half(1)


_conv_call = functools.partial(
    pl.kernel,
    out_type=tuple(jax.ShapeDtypeStruct((NP, DH), jnp.float32)
                   for _ in range(R * 2)),
    mesh=_mesh,
    scratch_types=[
        pltpu.VMEM_SHARED((NP, DH), jnp.float32),
        pltpu.VMEM((G, CH), jnp.int32),
        pltpu.VMEM((G, CH), jnp.int32),
        pltpu.VMEM((CH, DH), jnp.float32),
        pltpu.VMEM((CH, DH), jnp.float32),
        pltpu.SemaphoreType.DMA,
        pltpu.SemaphoreType.DMA,
    ],
)(_conv_body)


def _mm_body(x_ref, w_ref, degc_ref, *out_refs):
    xb = x_ref[...]
    dis = lax.rsqrt(degc_ref[...] + 1.0)
    for r in range(R):
        h = jnp.dot(xb, w_ref[r], preferred_element_type=jnp.float32)
        hs = h * dis[:, r][:, None]
        out_refs[2 * r][...] = hs[:, :DH]
        out_refs[2 * r + 1][...] = hs[:, DH:]


def _fuse_body(a00, a01, a10, a11, a20, a21, degc_ref, bias_ref, w_ref,
               *out_refs):
    a_refs = ((a00, a01), (a10, a11), (a20, a21))
    dis = lax.rsqrt(degc_ref[...] + 1.0)
    xs = None
    for r in range(R):
        o = jnp.concatenate([a_refs[r][0][...], a_refs[r][1][...]], axis=1)
        o = o * dis[:, r][:, None] + bias_ref[r][None, :]
        o = jnp.maximum(o, 0.0)
        xs = o if xs is None else jnp.maximum(xs, o)
    for r in range(R):
        h = jnp.dot(xs, w_ref[r], preferred_element_type=jnp.float32)
        hs = h * dis[:, r][:, None]
        out_refs[2 * r][...] = hs[:, :DH]
        out_refs[2 * r + 1][...] = hs[:, DH:]


def _tail_body(a00, a01, a10, a11, a20, a21, degc_ref, bias_ref, out_ref):
    a_refs = ((a00, a01), (a10, a11), (a20, a21))
    dis = lax.rsqrt(degc_ref[...] + 1.0)
    xs = None
    for r in range(R):
        o = jnp.concatenate([a_refs[r][0][...], a_refs[r][1][...]], axis=1)
        o = o * dis[:, r][:, None] + bias_ref[r][None, :]
        o = jnp.maximum(o, 0.0)
        xs = o if xs is None else jnp.maximum(xs, o)
    out_ref[...] = xs


_half_spec = pl.BlockSpec((BLK, DH), lambda i: (i, 0))
_degc_spec = pl.BlockSpec((BLK, R), lambda i: (i, 0))
_w_spec = pl.BlockSpec((R, D, D), lambda i: (0, 0, 0))
_b_spec = pl.BlockSpec((R, D), lambda i: (0, 0))

_mm_call = pl.pallas_call(
    _mm_body,
    grid=(GRID,),
    in_specs=[pl.BlockSpec((BLK, D), lambda i: (i, 0)), _w_spec, _degc_spec],
    out_specs=tuple(_half_spec for _ in range(R * 2)),
    out_shape=tuple(jax.ShapeDtypeStruct((NP, DH), jnp.float32)
                    for _ in range(R * 2)),
)

_fuse_call = pl.pallas_call(
    _fuse_body,
    grid=(GRID,),
    in_specs=[_half_spec] * (R * 2) + [_degc_spec, _b_spec, _w_spec],
    out_specs=tuple(_half_spec for _ in range(R * 2)),
    out_shape=tuple(jax.ShapeDtypeStruct((NP, DH), jnp.float32)
                    for _ in range(R * 2)),
)

_tail_call = pl.pallas_call(
    _tail_body,
    grid=(GRID,),
    in_specs=[_half_spec] * (R * 2) + [_degc_spec, _b_spec],
    out_specs=pl.BlockSpec((BLK, D), lambda i: (i, 0)),
    out_shape=jax.ShapeDtypeStruct((NP, D), jnp.float32),
)


def kernel(x, edge_index_list, W, b):
    ei = edge_index_list.astype(jnp.int32)
    pad_e = EPAD - E
    srcs = []
    dsts = []
    for r in range(R):
        sr = jnp.concatenate([ei[r, 0], jnp.zeros((pad_e,), jnp.int32)])
        dr = jnp.concatenate([ei[r, 1], jnp.full((pad_e,), N, jnp.int32)])
        srcs.append(sr.reshape(NS, NCH, CH))
        dsts.append(dr.reshape(NS, NCH, CH))

    deg = _deg_call(dsts[0], dsts[1], dsts[2])
    degc = jnp.transpose(deg)

    x_pad = jnp.pad(x, ((0, NP - N), (0, 0)))
    hs = _mm_call(x_pad, W[0], degc)
    acc0 = _conv_call(*hs, *srcs, *dsts)
    hs1 = _fuse_call(*acc0, degc, b[0], W[1])
    acc1 = _conv_call(*hs1, *srcs, *dsts)
    return _tail_call(*acc1, degc, b[1])[:N]


# speedup vs baseline: 7.3379x; 1.0863x over previous
import functools

import jax
import jax.numpy as jnp
from jax import lax
from jax.experimental import pallas as pl
from jax.experimental.pallas import tpu as pltpu
from jax.experimental.pallas import tpu_sc as plsc

N = 10000
NP = 10240
E = 160000
R = 3
D = 256
DH = 128
NC = 2
NS = 16
CH = 128
NCH = 80
HC = NCH // 2
EPT = CH * NCH
EPAD = NS * EPT
ROWS_PT = NP // NS
BLK = 2048
GRID = NP // BLK

_mesh = plsc.VectorSubcoreMesh(
    core_axis_name="c", subcore_axis_name="s", num_cores=NC, num_subcores=NS)


RPT = NP // NS


def _deg_body(d0, d1, d2, deg_out, hist_sh, idx_v, hist_v, comb_v, out_v):
    c = lax.axis_index("c")
    s = lax.axis_index("s")
    zeros16 = jnp.zeros((16,), jnp.float32)
    ones16 = jnp.ones((16,), jnp.float32)
    zidx16 = jnp.zeros((16,), jnp.int32)

    def do_rel(dref, r):
        pltpu.sync_copy(dref.at[s], idx_v)

        def zero(i, carry):
            hist_v[0, pl.ds(i * 16, 16)] = zeros16
            return carry

        lax.fori_loop(0, NP // 16, zero, 0)

        def acc_chunk(j, carry):
            def acc(k, carry2):
                iv = idx_v[j, pl.ds(k * 16, 16)]
                plsc.addupdate_scatter(hist_v, [zidx16, iv], ones16)
                return carry2

            return lax.fori_loop(0, CH // 16, acc, carry)

        lax.fori_loop(0, NCH, acc_chunk, 0)
        pltpu.sync_copy(hist_v, hist_sh.at[pl.ds(s, 1), :])
        plsc.subcore_barrier()
        pltpu.sync_copy(hist_sh.at[:, pl.ds(s * RPT, RPT)], comb_v)

        def comb(i, carry):
            tot = comb_v[0, pl.ds(i * 16, 16)]
            for t in range(1, NS):
                tot = tot + comb_v[t, pl.ds(i * 16, 16)]
            out_v[0, pl.ds(i * 16, 16)] = tot
            return carry

        lax.fori_loop(0, RPT // 16, comb, 0)
        pltpu.sync_copy(out_v, deg_out.at[pl.ds(r, 1), pl.ds(s * RPT, RPT)])
        plsc.subcore_barrier()

    @pl.when(c == 0)
    def _():
        do_rel(d0, 0)
        do_rel(d1, 1)

    @pl.when(c == 1)
    def _():
        do_rel(d2, 2)


_deg_call = functools.partial(
    pl.kernel,
    out_type=jax.ShapeDtypeStruct((R, NP), jnp.float32),
    mesh=_mesh,
    compiler_params=pltpu.CompilerParams(needs_layout_passes=False),
    scratch_types=[
        pltpu.VMEM_SHARED((NS, NP), jnp.float32),
        pltpu.VMEM((NCH, CH), jnp.int32),
        pltpu.VMEM((1, NP), jnp.float32),
        pltpu.VMEM((NS, RPT), jnp.float32),
        pltpu.VMEM((1, RPT), jnp.float32),
    ],
)(_deg_body)


def _conv_body(h00, h01, h10, h11, h20, h21,
               s0, s1, s2, d0, d1, d2,
               o00, o01, o10, o11, o20, o21,
               acc_sh, src_v, dst_v, buf0, buf1, semg0, semg1, sems0,
               sems1):
    c = lax.axis_index("c")
    s = lax.axis_index("s")
    h_refs = ((h00, h01), (h10, h11), (h20, h21))
    o_refs = ((o00, o01), (o10, o11), (o20, o21))
    s_refs = (s0, s1, s2)
    d_refs = (d0, d1, d2)

    def do_half(half):
        for r in range(R):
            h = h_refs[r][half]
            o = o_refs[r][half]
            pltpu.sync_copy(h.at[pl.ds(s * ROWS_PT, ROWS_PT)],
                            acc_sh.at[pl.ds(s * ROWS_PT, ROWS_PT)])
            plsc.subcore_barrier()

            npair = HC // 2
            for ih in range(2):
                pltpu.sync_copy(s_refs[r].at[s, pl.ds(ih * HC, HC)], src_v)
                pltpu.sync_copy(d_refs[r].at[s, pl.ds(ih * HC, HC)], dst_v)
                pltpu.async_copy(h.at[src_v.at[0]], buf0, semg0)
                pltpu.async_copy(h.at[src_v.at[1]], buf1, semg1)

                def body(j, carry):
                    i0 = 2 * j
                    pltpu.make_async_copy(h.at[src_v.at[i0]], buf0,
                                          semg0).wait()
                    pltpu.async_copy(buf0, acc_sh.at[dst_v.at[i0]], sems0,
                                     add=True)
                    pltpu.make_async_copy(h.at[src_v.at[i0 + 1]], buf1,
                                          semg1).wait()
                    pltpu.async_copy(buf1, acc_sh.at[dst_v.at[i0 + 1]], sems1,
                                     add=True)

                    @pl.when(j < npair - 1)
                    def _():
                        pltpu.make_async_copy(buf0, acc_sh.at[dst_v.at[i0]],
                                              sems0).wait()
                        pltpu.async_copy(h.at[src_v.at[i0 + 2]], buf0, semg0)
                        pltpu.make_async_copy(buf1,
                                              acc_sh.at[dst_v.at[i0 + 1]],
                                              sems1).wait()
                        pltpu.async_copy(h.at[src_v.at[i0 + 3]], buf1, semg1)

                    return carry

                lax.fori_loop(0, npair, body, 0)
                pltpu.make_async_copy(buf0, acc_sh.at[dst_v.at[HC - 2]],
                                      sems0).wait()
                pltpu.make_async_copy(buf1, acc_sh.at[dst_v.at[HC - 1]],
                                      sems1).wait()
            plsc.subcore_barrier()
            pltpu.sync_copy(acc_sh.at[pl.ds(s * ROWS_PT, ROWS_PT)],
                            o.at[pl.ds(s * ROWS_PT, ROWS_PT)])
            plsc.subcore_barrier()

    @pl.when(c == 0)
    def _():
        do_half(0)

    @pl.when(c == 1)
    def _():
        do_half(1)


_conv_call = functools.partial(
    pl.kernel,
    out_type=tuple(jax.ShapeDtypeStruct((NP, DH), jnp.float32)
                   for _ in range(R * 2)),
    mesh=_mesh,
    scratch_types=[
        pltpu.VMEM_SHARED((NP, DH), jnp.float32),
        pltpu.VMEM((HC, CH), jnp.int32),
        pltpu.VMEM((HC, CH), jnp.int32),
        pltpu.VMEM((CH, DH), jnp.float32),
        pltpu.VMEM((CH, DH), jnp.float32),
        pltpu.SemaphoreType.DMA,
        pltpu.SemaphoreType.DMA,
        pltpu.SemaphoreType.DMA,
        pltpu.SemaphoreType.DMA,
    ],
)(_conv_body)


def _mm_body(x_ref, w_ref, degc_ref, *out_refs):
    xb = x_ref[...]
    dis = lax.rsqrt(degc_ref[...] + 1.0)
    for r in range(R):
        h = jnp.dot(xb, w_ref[r], preferred_element_type=jnp.float32)
        hs = h * dis[:, r][:, None]
        out_refs[2 * r][...] = hs[:, :DH]
        out_refs[2 * r + 1][...] = hs[:, DH:]


def _fuse_body(a00, a01, a10, a11, a20, a21, degc_ref, bias_ref, w_ref,
               *out_refs):
    a_refs = ((a00, a01), (a10, a11), (a20, a21))
    dis = lax.rsqrt(degc_ref[...] + 1.0)
    xs = None
    for r in range(R):
        o = jnp.concatenate([a_refs[r][0][...], a_refs[r][1][...]], axis=1)
        o = o * dis[:, r][:, None] + bias_ref[r][None, :]
        o = jnp.maximum(o, 0.0)
        xs = o if xs is None else jnp.maximum(xs, o)
    for r in range(R):
        h = jnp.dot(xs, w_ref[r], preferred_element_type=jnp.float32)
        hs = h * dis[:, r][:, None]
        out_refs[2 * r][...] = hs[:, :DH]
        out_refs[2 * r + 1][...] = hs[:, DH:]


def _tail_body(a00, a01, a10, a11, a20, a21, degc_ref, bias_ref, out_ref):
    a_refs = ((a00, a01), (a10, a11), (a20, a21))
    dis = lax.rsqrt(degc_ref[...] + 1.0)
    xs = None
    for r in range(R):
        o = jnp.concatenate([a_refs[r][0][...], a_refs[r][1][...]], axis=1)
        o = o * dis[:, r][:, None] + bias_ref[r][None, :]
        o = jnp.maximum(o, 0.0)
        xs = o if xs is None else jnp.maximum(xs, o)
    out_ref[...] = xs


_half_spec = pl.BlockSpec((BLK, DH), lambda i: (i, 0))
_degc_spec = pl.BlockSpec((BLK, R), lambda i: (i, 0))
_w_spec = pl.BlockSpec((R, D, D), lambda i: (0, 0, 0))
_b_spec = pl.BlockSpec((R, D), lambda i: (0, 0))

_mm_call = pl.pallas_call(
    _mm_body,
    grid=(GRID,),
    in_specs=[pl.BlockSpec((BLK, D), lambda i: (i, 0)), _w_spec, _degc_spec],
    out_specs=tuple(_half_spec for _ in range(R * 2)),
    out_shape=tuple(jax.ShapeDtypeStruct((NP, DH), jnp.float32)
                    for _ in range(R * 2)),
)

_fuse_call = pl.pallas_call(
    _fuse_body,
    grid=(GRID,),
    in_specs=[_half_spec] * (R * 2) + [_degc_spec, _b_spec, _w_spec],
    out_specs=tuple(_half_spec for _ in range(R * 2)),
    out_shape=tuple(jax.ShapeDtypeStruct((NP, DH), jnp.float32)
                    for _ in range(R * 2)),
)

_tail_call = pl.pallas_call(
    _tail_body,
    grid=(GRID,),
    in_specs=[_half_spec] * (R * 2) + [_degc_spec, _b_spec],
    out_specs=pl.BlockSpec((BLK, D), lambda i: (i, 0)),
    out_shape=jax.ShapeDtypeStruct((NP, D), jnp.float32),
)


def kernel(x, edge_index_list, W, b):
    ei = edge_index_list.astype(jnp.int32)
    pad_e = EPAD - E
    srcs = []
    dsts = []
    for r in range(R):
        sr = jnp.concatenate([ei[r, 0], jnp.zeros((pad_e,), jnp.int32)])
        dr = jnp.concatenate([ei[r, 1], jnp.full((pad_e,), N, jnp.int32)])
        srcs.append(sr.reshape(NS, NCH, CH))
        dsts.append(dr.reshape(NS, NCH, CH))

    deg = _deg_call(dsts[0], dsts[1], dsts[2])
    degc = jnp.transpose(deg)

    x_pad = jnp.pad(x, ((0, NP - N), (0, 0)))
    hs = _mm_call(x_pad, W[0], degc)
    acc0 = _conv_call(*hs, *srcs, *dsts)
    hs1 = _fuse_call(*acc0, degc, b[0], W[1])
    acc1 = _conv_call(*hs1, *srcs, *dsts)
    return _tail_call(*acc1, degc, b[1])[:N]
